# baseline (speedup 1.0000x reference)
import jax
import jax.numpy as jnp
from jax.experimental import pallas as pl
from jax.experimental.pallas import tpu as pltpu

B, S, L = 4, 4096, 2048
BS = 1024
SB = 32


def _scan_block(x_ref, o_ref, carry_ref):
    j = pl.program_id(1)

    @pl.when(j == 0)
    def _():
        carry_ref[...] = jnp.zeros_like(carry_ref)

    r = jax.lax.broadcasted_iota(jnp.int32, (SB, SB), 0)
    c = jax.lax.broadcasted_iota(jnp.int32, (SB, SB), 1)
    tri = (r > c).astype(jnp.bfloat16)

    carry = carry_ref[...]
    for k in range(BS // SB):
        sub = x_ref[0, k * SB:(k + 1) * SB, :]
        excl = jax.lax.dot(
            tri, sub.astype(jnp.bfloat16),
            preferred_element_type=jnp.float32,
        )
        o_ref[0, k * SB:(k + 1) * SB, :] = excl + carry
        carry = carry + jnp.sum(sub, axis=0, keepdims=True)
    carry_ref[...] = carry


@jax.jit
def kernel(x):
    grid = (B, S // BS)
    return pl.pallas_call(
        _scan_block,
        grid=grid,
        in_specs=[pl.BlockSpec((1, BS, L), lambda b, j: (b, j, 0))],
        out_specs=pl.BlockSpec((1, BS, L), lambda b, j: (b, j, 0)),
        out_shape=jax.ShapeDtypeStruct((B, S, L), jnp.float32),
        scratch_shapes=[pltpu.VMEM((1, L), jnp.float32)],
    )(x)


# speedup vs baseline: 1.0069x; 1.0069x over previous
import jax
import jax.numpy as jnp
from jax.experimental import pallas as pl
from jax.experimental.pallas import tpu as pltpu

B, S, L = 4, 4096, 2048
BS = 1024
SB = 64


def _scan_block(x_ref, o_ref, carry_ref):
    j = pl.program_id(1)

    @pl.when(j == 0)
    def _():
        carry_ref[...] = jnp.zeros_like(carry_ref)

    r = jax.lax.broadcasted_iota(jnp.int32, (SB, SB), 0)
    c = jax.lax.broadcasted_iota(jnp.int32, (SB, SB), 1)
    tri = (r > c).astype(jnp.bfloat16)

    carry = carry_ref[...]
    for k in range(BS // SB):
        sub = x_ref[0, k * SB:(k + 1) * SB, :]
        excl = jax.lax.dot(
            tri, sub.astype(jnp.bfloat16),
            preferred_element_type=jnp.float32,
        )
        o_ref[0, k * SB:(k + 1) * SB, :] = excl + carry
        carry = carry + jnp.sum(sub, axis=0, keepdims=True)
    carry_ref[...] = carry


@jax.jit
def kernel(x):
    grid = (B, S // BS)
    return pl.pallas_call(
        _scan_block,
        grid=grid,
        in_specs=[pl.BlockSpec((1, BS, L), lambda b, j: (b, j, 0))],
        out_specs=pl.BlockSpec((1, BS, L), lambda b, j: (b, j, 0)),
        out_shape=jax.ShapeDtypeStruct((B, S, L), jnp.float32),
        scratch_shapes=[pltpu.VMEM((1, L), jnp.float32)],
    )(x)
